# baseline (speedup 1.0000x reference)
import functools

import jax
import jax.numpy as jnp
from jax import lax
from jax.experimental import pallas as pl
from jax.experimental.pallas import tpu as pltpu
from jax.experimental.pallas import tpu_sc as plsc

BATCH = 16384
BAG = 50
DIM = 64
NC = 2
NS = 16
NW = NC * NS
BPW = BATCH // NW
CB = 1
NBUF = 8
RPG = CB * BAG
NCHUNK = BPW // CB
LANES = 16


def _bag_sum(rows_ref, row_base):
    init = tuple(jnp.zeros((LANES,), jnp.float32) for _ in range(4))

    @plsc.parallel_loop(0, BAG, unroll=10, carry=init)
    def body(r, acc):
        row = row_base + r
        return tuple(
            acc[h] + rows_ref[row, pl.ds(LANES * h, LANES)] for h in range(4)
        )

    return body


def _embedding_bag_mean(lang3, table):
    mesh = plsc.VectorSubcoreMesh(core_axis_name="c", subcore_axis_name="s")

    @functools.partial(
        pl.kernel,
        out_type=jax.ShapeDtypeStruct((BATCH, DIM), jnp.float32),
        mesh=mesh,
        compiler_params=pltpu.CompilerParams(
            use_tc_tiling_on_sc=False, needs_layout_passes=False),
        scratch_types=[
            pltpu.VMEM((BPW, BAG), jnp.int32),
            *[pltpu.VMEM((RPG, DIM), jnp.float32) for _ in range(NBUF)],
            pltpu.VMEM((BPW, DIM), jnp.float32),
            *[pltpu.SemaphoreType.DMA for _ in range(NBUF)],
        ],
    )
    def kern(lang_hbm, table_hbm, out_hbm, idx_all, *rest):
        rows = rest[:NBUF]
        out_v = rest[NBUF]
        sems = rest[NBUF + 1:]
        wid = lax.axis_index("s") * NC + lax.axis_index("c")
        scale = jnp.float32(1.0 / BAG)

        pltpu.sync_copy(lang_hbm.at[pl.ds(wid * BPW, BPW)], idx_all)

        def idx_chunk(g):
            return idx_all.at[g]

        for b in range(NBUF):
            pltpu.async_copy(table_hbm.at[idx_chunk(b)], rows[b], sems[b])

        def chunk_body(i, _):
            for b in range(NBUF):
                g = NBUF * i + b
                pltpu.make_async_copy(
                    table_hbm.at[idx_chunk(0)], rows[b], sems[b]
                ).wait()
                for c in range(CB):
                    acc = _bag_sum(rows[b], c * BAG)
                    bag = g * CB + c
                    for h in range(4):
                        out_v[bag, pl.ds(LANES * h, LANES)] = acc[h] * scale

                @pl.when(g + NBUF < NCHUNK)
                def _():
                    pltpu.async_copy(
                        table_hbm.at[idx_chunk(g + NBUF)], rows[b], sems[b]
                    )
            return 0

        lax.fori_loop(0, NCHUNK // NBUF, chunk_body, 0)

        pltpu.sync_copy(out_v, out_hbm.at[pl.ds(wid * BPW, BPW)])

    return kern(lang3, table)


def kernel(lang, table):
    return _embedding_bag_mean(lang.astype(jnp.int32), table)


# speedup vs baseline: 1.0096x; 1.0096x over previous
import functools

import jax
import jax.numpy as jnp
from jax import lax
from jax.experimental import pallas as pl
from jax.experimental.pallas import tpu as pltpu
from jax.experimental.pallas import tpu_sc as plsc

BATCH = 16384
BAG = 50
DIM = 64
NC = 2
NS = 16
NW = NC * NS
BPW = BATCH // NW
CB = 4
NBUF = 4
RPG = CB * BAG
NCHUNK = BPW // CB
LANES = 16


def _bag_sum(rows_ref, row_base):
    init = tuple(jnp.zeros((LANES,), jnp.float32) for _ in range(4))

    @plsc.parallel_loop(0, BAG, unroll=10, carry=init)
    def body(r, acc):
        row = row_base + r
        return tuple(
            acc[h] + rows_ref[row, pl.ds(LANES * h, LANES)] for h in range(4)
        )

    return body


def _embedding_bag_mean(lang3, table):
    mesh = plsc.VectorSubcoreMesh(core_axis_name="c", subcore_axis_name="s")

    @functools.partial(
        pl.kernel,
        out_type=jax.ShapeDtypeStruct((BATCH * DIM,), jnp.float32),
        mesh=mesh,
        compiler_params=pltpu.CompilerParams(
            use_tc_tiling_on_sc=False, needs_layout_passes=False),
        scratch_types=[
            pltpu.VMEM((BPW * BAG,), jnp.int32),
            *[pltpu.VMEM((RPG, DIM), jnp.float32) for _ in range(NBUF)],
            pltpu.VMEM((BPW * DIM,), jnp.float32),
            *[pltpu.SemaphoreType.DMA for _ in range(NBUF)],
        ],
    )
    def kern(lang_hbm, table_hbm, out_hbm, idx_all, *rest):
        rows = rest[:NBUF]
        out_v = rest[NBUF]
        sems = rest[NBUF + 1:]
        wid = lax.axis_index("s") * NC + lax.axis_index("c")
        scale = jnp.float32(1.0 / BAG)

        pltpu.sync_copy(
            lang_hbm.at[pl.ds(wid * BPW * BAG, BPW * BAG)], idx_all)

        def idx_chunk(g):
            return idx_all.at[pl.ds(g * RPG, RPG)]

        for b in range(NBUF):
            pltpu.async_copy(table_hbm.at[idx_chunk(b)], rows[b], sems[b])

        def chunk_body(i, _):
            for b in range(NBUF):
                g = NBUF * i + b
                pltpu.make_async_copy(
                    table_hbm.at[idx_chunk(0)], rows[b], sems[b]
                ).wait()
                for c in range(CB):
                    acc = _bag_sum(rows[b], c * BAG)
                    bag = g * CB + c
                    for h in range(4):
                        out_v[pl.ds(bag * DIM + LANES * h, LANES)] = (
                            acc[h] * scale)

                @pl.when(g + NBUF < NCHUNK)
                def _():
                    pltpu.async_copy(
                        table_hbm.at[idx_chunk(g + NBUF)], rows[b], sems[b]
                    )
            return 0

        lax.fori_loop(0, NCHUNK // NBUF, chunk_body, 0)

        pltpu.sync_copy(
            out_v, out_hbm.at[pl.ds(wid * BPW * DIM, BPW * DIM)])

    return kern(lang3, table)


def kernel(lang, table):
    flat = lang.astype(jnp.int32).reshape(BATCH * BAG)
    return _embedding_bag_mean(flat, table).reshape(BATCH, DIM)
